# baseline (speedup 1.0000x reference)
import functools

import jax
import jax.numpy as jnp
from jax import lax
from jax.experimental import pallas as pl
from jax.experimental.pallas import tpu as pltpu
from jax.experimental.pallas import tpu_sc as plsc

T = 2048
D = 768
E = 8
F = 2048
CAP = 768
NSLOT = E * CAP
EPS = 1e-6
NW = 32


def _route_body(x_ref, lnw_ref, wg_ref, nx_ref, slots_ref, inv_ref, wslot_ref):
    x = x_ref[...]
    var = jnp.mean(x * x, axis=1, keepdims=True)
    nx = x * lax.rsqrt(var + EPS) * lnw_ref[...]
    nx_ref[...] = nx

    logits = jnp.dot(nx, wg_ref[...], preferred_element_type=jnp.float32)
    lane = lax.broadcasted_iota(jnp.int32, (T, 128), 1)
    neg = jnp.float32(-1e30)
    logits = jnp.where(lane < E, logits, neg)
    m1 = jnp.max(logits, axis=1, keepdims=True)
    e1 = jnp.min(jnp.where(logits == m1, lane, 128), axis=1, keepdims=True)
    l2 = jnp.where(lane == e1, neg, logits)
    m2 = jnp.max(l2, axis=1, keepdims=True)
    e2 = jnp.min(jnp.where(l2 == m2, lane, 128), axis=1, keepdims=True)
    ed = jnp.exp(m2 - m1)
    w1 = 1.0 / (1.0 + ed)
    w2 = 1.0 - w1

    oh = ((lane == e1) | (lane == e2)).astype(jnp.float32)
    r128 = lax.broadcasted_iota(jnp.int32, (128, 128), 0)
    c128 = lax.broadcasted_iota(jnp.int32, (128, 128), 1)
    tri_s = (c128 < r128).astype(jnp.float32)
    r16 = lax.broadcasted_iota(jnp.int32, (16, 16), 0)
    c16 = lax.broadcasted_iota(jnp.int32, (16, 16), 1)
    tri16 = (c16 < r16).astype(jnp.float32)
    blocks, sums = [], []
    for b in range(16):
        ob = oh[b * 128:(b + 1) * 128, :]
        blocks.append(jnp.dot(tri_s, ob, preferred_element_type=jnp.float32))
        sums.append(jnp.sum(ob, axis=0, keepdims=True))
    bsums = jnp.concatenate(sums, axis=0)
    carry = jnp.dot(tri16, bsums, preferred_element_type=jnp.float32)
    cex = jnp.concatenate(
        [blocks[b] + carry[b:b + 1, :] for b in range(16)], axis=0)

    oh1 = (lane == e1).astype(jnp.float32)
    oh2 = (lane == e2).astype(jnp.float32)
    pos1 = jnp.sum(cex * oh1, axis=1, keepdims=True).astype(jnp.int32)
    pos2 = jnp.sum(cex * oh2, axis=1, keepdims=True).astype(jnp.int32)
    v1 = pos1 < CAP
    v2 = pos2 < CAP
    s1r = jnp.where(v1, e1 * CAP + pos1, NSLOT)
    s2r = jnp.where(v2, e2 * CAP + pos2, NSLOT)
    lane1 = lax.broadcasted_iota(jnp.int32, (1, 128), 1)
    cnt = jnp.sum(oh, axis=0, keepdims=True)
    cntm = jnp.where(lane1 < E, cnt, jnp.float32(1e9))
    cmin = jnp.min(cntm, axis=1, keepdims=True)
    emin = jnp.min(jnp.where(cntm == cmin, lane1, 128), axis=1, keepdims=True)
    dropslot = emin * CAP + (CAP - 1)
    s1c = jnp.where(v1, s1r, dropslot)
    s2c = jnp.where(v2, s2r, dropslot)
    w1m = jnp.where(v1, w1, 0.0)
    w2m = jnp.where(v2, w2, 0.0)

    slots_ref[...] = jnp.where(
        lane == 0, s1r,
        jnp.where(lane == 1, s2r,
                  jnp.where(lane == 2, s1c,
                            jnp.where(lane == 3, s2c, 0))))

    tok = lax.broadcasted_iota(jnp.int32, (T, 128), 0).astype(jnp.float32)

    def inv_body(r, _):
        c = lane1 + r * 128
        ma = (s1r == c).astype(jnp.float32)
        mb = (s2r == c).astype(jnp.float32)
        m = ma + mb
        invv = jnp.sum(m * tok, axis=0, keepdims=True)
        filled = jnp.sum(m, axis=0, keepdims=True)
        inv_ref[pl.ds(r, 1), :] = jnp.where(
            filled > 0, invv.astype(jnp.int32), c % T)
        wslot_ref[pl.ds(r, 1), :] = jnp.sum(
            ma * w1m + mb * w2m, axis=0, keepdims=True)
        return 0

    lax.fori_loop(0, 48, inv_body, 0)


_route = pl.pallas_call(
    _route_body,
    out_shape=(
        jax.ShapeDtypeStruct((T, D), jnp.float32),
        jax.ShapeDtypeStruct((T, 128), jnp.int32),
        jax.ShapeDtypeStruct((48, 128), jnp.int32),
        jax.ShapeDtypeStruct((48, 128), jnp.float32),
    ),
)


_DCHUNK = 96


@functools.cache
def _make_dispatch():
    mesh = plsc.VectorSubcoreMesh(core_axis_name="c", subcore_axis_name="s")

    @functools.partial(
        pl.kernel,
        mesh=mesh,
        out_type=jax.ShapeDtypeStruct((NSLOT, D), jnp.float32),
        scratch_types=[
            pltpu.VMEM((_DCHUNK,), jnp.int32),
            pltpu.VMEM((_DCHUNK, D), jnp.float32),
            pltpu.SemaphoreType.DMA,
        ],
    )
    def dispatch(nx_hbm, inv_hbm, ebuf_hbm, idx_v, rows_v, sem):
        wid = lax.axis_index("s") * 2 + lax.axis_index("c")
        for chunk in range(NSLOT // (NW * _DCHUNK)):
            base = wid * (NSLOT // NW) + chunk * _DCHUNK
            pltpu.sync_copy(inv_hbm.at[pl.ds(base, _DCHUNK)], idx_v)
            pltpu.async_copy(nx_hbm.at[idx_v], rows_v, sem).wait()
            pltpu.sync_copy(rows_v, ebuf_hbm.at[pl.ds(base, _DCHUNK)])

    return dispatch


def _expert_body(xe_ref, wi_ref, wo_ref, ws_ref, out_ref):
    wrow = jnp.concatenate(
        [ws_ref[0, r:r + 1, :] for r in range(CAP // 128)], axis=1)
    ri = lax.broadcasted_iota(jnp.int32, (CAP, CAP), 0)
    ci = lax.broadcasted_iota(jnp.int32, (CAP, CAP), 1)
    diag = ((ri == ci).astype(jnp.float32) * wrow).astype(jnp.bfloat16)
    xs = jnp.dot(diag, xe_ref[0].astype(jnp.bfloat16),
                 preferred_element_type=jnp.float32).astype(jnp.bfloat16)
    h = jnp.maximum(
        jnp.dot(xs, wi_ref[0].astype(jnp.bfloat16),
                preferred_element_type=jnp.float32), 0.0)
    out_ref[0] = jnp.dot(h.astype(jnp.bfloat16),
                         wo_ref[0].astype(jnp.bfloat16),
                         preferred_element_type=jnp.float32)


_experts = pl.pallas_call(
    _expert_body,
    grid=(E,),
    in_specs=[
        pl.BlockSpec((1, CAP, D), lambda e: (e, 0, 0)),
        pl.BlockSpec((1, D, F), lambda e: (e, 0, 0)),
        pl.BlockSpec((1, F, D), lambda e: (e, 0, 0)),
        pl.BlockSpec((1, CAP // 128, 128), lambda e: (e, 0, 0)),
    ],
    out_specs=pl.BlockSpec((1, CAP, D), lambda e: (e, 0, 0)),
    out_shape=jax.ShapeDtypeStruct((E, CAP, D), jnp.float32),
)


_CCHUNK = 64


@functools.cache
def _make_combine():
    mesh = plsc.VectorSubcoreMesh(core_axis_name="c", subcore_axis_name="s")

    @functools.partial(
        pl.kernel,
        mesh=mesh,
        out_type=jax.ShapeDtypeStruct((2 * T, D), jnp.float32),
        scratch_types=[
            pltpu.VMEM((_CCHUNK,), jnp.int32),
            pltpu.VMEM((_CCHUNK, D), jnp.float32),
            pltpu.SemaphoreType.DMA,
        ],
    )
    def combine(eo_hbm, scidx_hbm, g_hbm, idx_v, rows_v, sem):
        wid = lax.axis_index("s") * 2 + lax.axis_index("c")
        base = wid * _CCHUNK
        for j in range(2):
            pltpu.sync_copy(scidx_hbm.at[pl.ds(j * T + base, _CCHUNK)], idx_v)
            pltpu.async_copy(eo_hbm.at[idx_v], rows_v, sem).wait()
            pltpu.sync_copy(rows_v, g_hbm.at[pl.ds(j * T + base, _CCHUNK)])

    return combine


def _final_body(x_ref, g_ref, out_ref):
    out_ref[...] = x_ref[...] + g_ref[0:T, :] + g_ref[T:2 * T, :]


_final = pl.pallas_call(
    _final_body,
    out_shape=jax.ShapeDtypeStruct((T, D), jnp.float32),
)


def kernel(hidden_states, ln_w, Wg, Wi, Wo):
    x = hidden_states.reshape(T, D)
    wgp = jnp.pad(Wg, ((0, 0), (0, 128 - E)))
    nx, slots, inv, wslot = _route(x, ln_w.reshape(1, D), wgp)
    ebuf = _make_dispatch()(nx, inv.reshape(NSLOT))
    eo = _experts(ebuf.reshape(E, CAP, D), Wi, Wo,
                  wslot.reshape(E, CAP // 128, 128)).reshape(NSLOT, D)
    scidx = slots[:, 2:4].T.reshape(2 * T)
    g = _make_combine()(eo, scidx)
    y = _final(x, g)
    return y.reshape(1, T, D)


# speedup vs baseline: 1.2238x; 1.2238x over previous
import functools

import jax
import jax.numpy as jnp
from jax import lax
from jax.experimental import pallas as pl
from jax.experimental.pallas import tpu as pltpu
from jax.experimental.pallas import tpu_sc as plsc

T = 2048
D = 768
E = 8
F = 2048
CAP = 768
NSLOT = E * CAP
EPS = 1e-6
NW = 32


def _route_body(x_ref, lnw_ref, wg_ref, nx_ref, slots_ref, wts_ref, inv_ref):
    x = x_ref[...]
    var = jnp.mean(x * x, axis=1, keepdims=True)
    nx = x * lax.rsqrt(var + EPS) * lnw_ref[...]
    nx_ref[...] = nx

    logits = jnp.dot(nx, wg_ref[...], preferred_element_type=jnp.float32)
    lane = lax.broadcasted_iota(jnp.int32, (T, 128), 1)
    neg = jnp.float32(-1e30)
    logits = jnp.where(lane < E, logits, neg)
    m1 = jnp.max(logits, axis=1, keepdims=True)
    e1 = jnp.min(jnp.where(logits == m1, lane, 128), axis=1, keepdims=True)
    l2 = jnp.where(lane == e1, neg, logits)
    m2 = jnp.max(l2, axis=1, keepdims=True)
    e2 = jnp.min(jnp.where(l2 == m2, lane, 128), axis=1, keepdims=True)
    ed = jnp.exp(m2 - m1)
    w1 = 1.0 / (1.0 + ed)
    w2 = 1.0 - w1

    oh = ((lane == e1) | (lane == e2)).astype(jnp.float32)
    r128 = lax.broadcasted_iota(jnp.int32, (128, 128), 0)
    c128 = lax.broadcasted_iota(jnp.int32, (128, 128), 1)
    tri_s = (c128 < r128).astype(jnp.float32)
    r16 = lax.broadcasted_iota(jnp.int32, (16, 16), 0)
    c16 = lax.broadcasted_iota(jnp.int32, (16, 16), 1)
    tri16 = (c16 < r16).astype(jnp.float32)
    blocks, sums = [], []
    for b in range(16):
        ob = oh[b * 128:(b + 1) * 128, :]
        blocks.append(jnp.dot(tri_s, ob, preferred_element_type=jnp.float32))
        sums.append(jnp.sum(ob, axis=0, keepdims=True))
    bsums = jnp.concatenate(sums, axis=0)
    carry = jnp.dot(tri16, bsums, preferred_element_type=jnp.float32)
    cex = jnp.concatenate(
        [blocks[b] + carry[b:b + 1, :] for b in range(16)], axis=0)

    oh1 = (lane == e1).astype(jnp.float32)
    oh2 = (lane == e2).astype(jnp.float32)
    pos1 = jnp.sum(cex * oh1, axis=1, keepdims=True).astype(jnp.int32)
    pos2 = jnp.sum(cex * oh2, axis=1, keepdims=True).astype(jnp.int32)
    v1 = pos1 < CAP
    v2 = pos2 < CAP
    s1r = jnp.where(v1, e1 * CAP + pos1, NSLOT)
    s2r = jnp.where(v2, e2 * CAP + pos2, NSLOT)
    slot00 = e1[0:1, 0:1] * CAP
    s1c = jnp.where(v1, s1r, slot00)
    s2c = jnp.where(v2, s2r, slot00)
    w1m = jnp.where(v1, w1, 0.0)
    w2m = jnp.where(v2, w2, 0.0)

    slots_ref[...] = jnp.where(
        lane == 0, s1r,
        jnp.where(lane == 1, s2r,
                  jnp.where(lane == 2, s1c,
                            jnp.where(lane == 3, s2c, 0))))
    wts_ref[...] = jnp.where(lane == 0, w1m, jnp.where(lane == 1, w2m, 0.0))

    tok = lax.broadcasted_iota(jnp.int32, (T, 128), 0).astype(jnp.float32)
    inv_rows = []
    for r in range(48):
        c = lax.broadcasted_iota(jnp.int32, (1, 128), 1) + r * 128
        m = ((s1r == c) | (s2r == c)).astype(jnp.float32)
        invv = jnp.sum(m * tok, axis=0, keepdims=True)
        filled = jnp.sum(m, axis=0, keepdims=True)
        inv_rows.append(jnp.where(filled > 0, invv.astype(jnp.int32), c % T))
    inv_ref[...] = jnp.concatenate(inv_rows, axis=0)


_route = pl.pallas_call(
    _route_body,
    out_shape=(
        jax.ShapeDtypeStruct((T, D), jnp.float32),
        jax.ShapeDtypeStruct((T, 128), jnp.int32),
        jax.ShapeDtypeStruct((T, 128), jnp.float32),
        jax.ShapeDtypeStruct((48, 128), jnp.int32),
    ),
)


_DCHUNK = 64
_DROWS = NSLOT // NW


@functools.cache
def _make_dispatch():
    mesh = plsc.VectorSubcoreMesh(core_axis_name="c", subcore_axis_name="s")

    @functools.partial(
        pl.kernel,
        mesh=mesh,
        out_type=jax.ShapeDtypeStruct((NSLOT, D), jnp.float32),
        scratch_types=[
            pltpu.VMEM((_DROWS,), jnp.int32),
            pltpu.VMEM((_DCHUNK, D), jnp.float32),
            pltpu.VMEM((_DCHUNK, D), jnp.float32),
            pltpu.SemaphoreType.DMA,
            pltpu.SemaphoreType.DMA,
            pltpu.SemaphoreType.DMA,
            pltpu.SemaphoreType.DMA,
        ],
    )
    def dispatch(nx_hbm, inv_hbm, ebuf_hbm,
                 idx_v, buf0, buf1, gs0, gs1, ws0, ws1):
        wid = lax.axis_index("s") * 2 + lax.axis_index("c")
        base = wid * _DROWS
        pltpu.sync_copy(inv_hbm.at[pl.ds(base, _DROWS)], idx_v)
        g0 = pltpu.async_copy(
            nx_hbm.at[idx_v.at[pl.ds(0, _DCHUNK)]], buf0, gs0)
        g1 = pltpu.async_copy(
            nx_hbm.at[idx_v.at[pl.ds(_DCHUNK, _DCHUNK)]], buf1, gs1)
        g0.wait()
        w0 = pltpu.async_copy(buf0, ebuf_hbm.at[pl.ds(base, _DCHUNK)], ws0)
        g1.wait()
        w1 = pltpu.async_copy(
            buf1, ebuf_hbm.at[pl.ds(base + _DCHUNK, _DCHUNK)], ws1)
        w0.wait()
        g2 = pltpu.async_copy(
            nx_hbm.at[idx_v.at[pl.ds(2 * _DCHUNK, _DCHUNK)]], buf0, gs0)
        g2.wait()
        w2 = pltpu.async_copy(
            buf0, ebuf_hbm.at[pl.ds(base + 2 * _DCHUNK, _DCHUNK)], ws0)
        w1.wait()
        w2.wait()

    return dispatch


def _expert_body(xe_ref, wi_ref, wo_ref, out_ref):
    xb = xe_ref[0].astype(jnp.bfloat16)
    h = jnp.maximum(
        jnp.dot(xb, wi_ref[0].astype(jnp.bfloat16),
                preferred_element_type=jnp.float32), 0.0)
    out_ref[0] = jnp.dot(h.astype(jnp.bfloat16),
                         wo_ref[0].astype(jnp.bfloat16),
                         preferred_element_type=jnp.float32)


_experts = pl.pallas_call(
    _expert_body,
    grid=(E,),
    in_specs=[
        pl.BlockSpec((1, CAP, D), lambda e: (e, 0, 0)),
        pl.BlockSpec((1, D, F), lambda e: (e, 0, 0)),
        pl.BlockSpec((1, F, D), lambda e: (e, 0, 0)),
    ],
    out_specs=pl.BlockSpec((1, CAP, D), lambda e: (e, 0, 0)),
    out_shape=jax.ShapeDtypeStruct((E, CAP, D), jnp.float32),
)


_CCHUNK = 64


@functools.cache
def _make_combine():
    mesh = plsc.VectorSubcoreMesh(core_axis_name="c", subcore_axis_name="s")

    @functools.partial(
        pl.kernel,
        mesh=mesh,
        out_type=jax.ShapeDtypeStruct((2 * T, D), jnp.float32),
        scratch_types=[
            pltpu.VMEM((2 * _CCHUNK,), jnp.int32),
            pltpu.VMEM((_CCHUNK, D), jnp.float32),
            pltpu.VMEM((_CCHUNK, D), jnp.float32),
            pltpu.SemaphoreType.DMA,
            pltpu.SemaphoreType.DMA,
            pltpu.SemaphoreType.DMA,
            pltpu.SemaphoreType.DMA,
        ],
    )
    def combine(eo_hbm, scidx_hbm, g_hbm,
                idx_v, buf0, buf1, gs0, gs1, ws0, ws1):
        wid = lax.axis_index("s") * 2 + lax.axis_index("c")
        base = wid * _CCHUNK
        pltpu.sync_copy(scidx_hbm.at[pl.ds(base, _CCHUNK)],
                        idx_v.at[pl.ds(0, _CCHUNK)])
        pltpu.sync_copy(scidx_hbm.at[pl.ds(T + base, _CCHUNK)],
                        idx_v.at[pl.ds(_CCHUNK, _CCHUNK)])
        g0 = pltpu.async_copy(
            eo_hbm.at[idx_v.at[pl.ds(0, _CCHUNK)]], buf0, gs0)
        g1 = pltpu.async_copy(
            eo_hbm.at[idx_v.at[pl.ds(_CCHUNK, _CCHUNK)]], buf1, gs1)
        g0.wait()
        w0 = pltpu.async_copy(buf0, g_hbm.at[pl.ds(base, _CCHUNK)], ws0)
        g1.wait()
        w1 = pltpu.async_copy(buf1, g_hbm.at[pl.ds(T + base, _CCHUNK)], ws1)
        w0.wait()
        w1.wait()

    return combine


def _final_body(x_ref, g_ref, wts_ref, out_ref):
    g1 = g_ref[0:T, :]
    g2 = g_ref[T:2 * T, :]
    w1 = wts_ref[:, 0:1]
    w2 = wts_ref[:, 1:2]
    out_ref[...] = x_ref[...] + w1 * g1 + w2 * g2


_final = pl.pallas_call(
    _final_body,
    out_shape=jax.ShapeDtypeStruct((T, D), jnp.float32),
)


def kernel(hidden_states, ln_w, Wg, Wi, Wo):
    x = hidden_states.reshape(T, D)
    wgp = jnp.pad(Wg, ((0, 0), (0, 128 - E)))
    nx, slots, wts, inv = _route(x, ln_w.reshape(1, D), wgp)
    ebuf = _make_dispatch()(nx, inv.reshape(NSLOT))
    eo = _experts(ebuf.reshape(E, CAP, D), Wi, Wo).reshape(NSLOT, D)
    scidx = slots[:, 2:4].T.reshape(2 * T)
    g = _make_combine()(eo, scidx)
    y = _final(x, g, wts)
    return y.reshape(1, T, D)
